# Initial kernel scaffold; baseline (speedup 1.0000x reference)
import jax
import jax.numpy as jnp
from jax.experimental import pallas as pl


def kernel(x, pos_table, gamma, beta):
    raise NotImplementedError("write your pallas kernel here")


# speedup vs baseline: 2.5866x; 2.5866x over previous
import jax
import jax.numpy as jnp
from jax.experimental import pallas as pl

_EPS = 1e-3


def _body(x_ref, pos_ref, g_ref, b_ref, o_ref):
    h = x_ref[...] + pos_ref[...][None]
    mean = jnp.mean(h, axis=-1, keepdims=True)
    d = h - mean
    var = jnp.mean(d * d, axis=-1, keepdims=True)
    o_ref[...] = d * jax.lax.rsqrt(var + _EPS) * g_ref[...] + b_ref[...]


def kernel(x, pos_table, gamma, beta):
    B, S, H = x.shape
    blk = 256
    out = pl.pallas_call(
        _body,
        grid=(S // blk,),
        in_specs=[
            pl.BlockSpec((B, blk, H), lambda j: (0, j, 0)),
            pl.BlockSpec((blk, H), lambda j: (j, 0)),
            pl.BlockSpec((1, H), lambda j: (0, 0)),
            pl.BlockSpec((1, H), lambda j: (0, 0)),
        ],
        out_specs=pl.BlockSpec((B, blk, H), lambda j: (0, j, 0)),
        out_shape=jax.ShapeDtypeStruct(x.shape, x.dtype),
    )(x, pos_table, gamma.reshape(1, H), beta.reshape(1, H))
    return out
